# baseline (speedup 1.0000x reference)
import functools

import jax
import jax.numpy as jnp
from jax.experimental import pallas as pl
from jax.experimental.pallas import tpu as pltpu

B, T, D = 16, 4096, 128
CH = 2048
SUB = 256
NCH = T // CH
NEG_INF = float("-inf")


def _body(lens_ref, x_hbm, wt_ref, b_ref, out_hbm, xbuf, obuf, sems, osems):
    j = pl.program_id(0)
    length = lens_ref[j]
    slot = j % 2

    def _sub_copy(r, s, k, sub):
        return pltpu.make_async_copy(
            x_hbm.at[r, pl.ds(k * CH + sub * SUB, SUB), :],
            xbuf.at[s, k, pl.ds(sub * SUB, SUB), :],
            sems.at[s, k],
        )

    def _start_row(r, s):
        row_len = lens_ref[r]
        for k in range(NCH):
            t0 = k * CH

            @pl.when(t0 + CH <= row_len)
            def _start_full():
                pltpu.make_async_copy(
                    x_hbm.at[r, pl.ds(t0, CH), :],
                    xbuf.at[s, k],
                    sems.at[s, k],
                ).start()

            @pl.when(jnp.logical_and(t0 < row_len, t0 + CH > row_len))
            def _start_partial():
                for sub in range(CH // SUB):
                    @pl.when(t0 + sub * SUB < row_len)
                    def _start_sub():
                        _sub_copy(r, s, k, sub).start()

    @pl.when(j == 0)
    def _prologue():
        _start_row(0, 0)

    @pl.when(j + 1 < B)
    def _prefetch_next():
        _start_row(j + 1, (j + 1) % 2)

    def _out_copy(r, s, k):
        return pltpu.make_async_copy(
            obuf.at[s, k],
            out_hbm.at[r, pl.ds(k * CH, CH), :],
            osems.at[s, k],
        )

    @pl.when(j >= 2)
    def _reclaim():
        for k in range(NCH):
            _out_copy(j - 2, slot, k).wait()

    def _dot(x):
        return (
            jnp.dot(x, wt_ref[...], preferred_element_type=jnp.float32)
            + b_ref[...]
        )

    def _wait_in(k):
        pltpu.make_async_copy(
            x_hbm.at[j, pl.ds(k * CH, CH), :],
            xbuf.at[slot, k],
            sems.at[slot, k],
        ).wait()

    for k in range(NCH):
        t0 = k * CH

        @pl.when(t0 + CH <= length)
        def _full_valid():
            _wait_in(k)
            obuf[slot, k] = _dot(xbuf[slot, k])

        @pl.when(t0 >= length)
        def _full_pad():
            obuf[slot, k] = jnp.full((CH, D), NEG_INF, dtype=jnp.float32)

        @pl.when(jnp.logical_and(t0 < length, t0 + CH > length))
        def _partial():
            for sub in range(CH // SUB):
                @pl.when(t0 + sub * SUB < length)
                def _wait_sub():
                    _sub_copy(j, slot, k, sub).wait()
            rows = t0 + jax.lax.broadcasted_iota(jnp.int32, (CH, D), 0)
            obuf[slot, k] = jnp.where(rows < length, _dot(xbuf[slot, k]), NEG_INF)

        _out_copy(j, slot, k).start()

    @pl.when(j == B - 1)
    def _epilogue():
        for k in range(NCH):
            _out_copy(j - 1, (j - 1) % 2, k).wait()
        for k in range(NCH):
            _out_copy(j, slot, k).wait()


@functools.partial(jax.jit, static_argnames=())
def _run(padded, lengths, wt, b2):
    grid_spec = pltpu.PrefetchScalarGridSpec(
        num_scalar_prefetch=1,
        grid=(B,),
        in_specs=[
            pl.BlockSpec(memory_space=pl.ANY),
            pl.BlockSpec((D, D), lambda j, lens: (0, 0)),
            pl.BlockSpec((1, D), lambda j, lens: (0, 0)),
        ],
        out_specs=pl.BlockSpec(memory_space=pl.ANY),
        scratch_shapes=[
            pltpu.VMEM((2, NCH, CH, D), jnp.float32),
            pltpu.VMEM((2, NCH, CH, D), jnp.float32),
            pltpu.SemaphoreType.DMA((2, NCH)),
            pltpu.SemaphoreType.DMA((2, NCH)),
        ],
    )
    out = pl.pallas_call(
        _body,
        grid_spec=grid_spec,
        out_shape=jax.ShapeDtypeStruct((B, T, D), jnp.float32),
        compiler_params=pltpu.CompilerParams(
            dimension_semantics=("arbitrary",),
        ),
    )(lengths, padded, wt, b2)
    return out


def kernel(padded, lengths, W, b):
    wt = W.T
    b2 = b.reshape(1, D)
    out = _run(padded, lengths.astype(jnp.int32), wt, b2)
    return out, lengths


# speedup vs baseline: 1.0186x; 1.0186x over previous
import functools

import jax
import jax.numpy as jnp
from jax.experimental import pallas as pl
from jax.experimental.pallas import tpu as pltpu

B, T, D = 16, 4096, 128
CH = 1024
SUB = 256
NCH = T // CH
NEG_INF = float("-inf")


def _body(lens_ref, x_hbm, wt_ref, b_ref, out_hbm, xbuf, obuf, cbuf, sems, osems):
    j = pl.program_id(0)
    length = lens_ref[j]
    slot = j % 2

    @pl.when(j == 0)
    def _init_const():
        cbuf[...] = jnp.full((CH, D), NEG_INF, dtype=jnp.float32)

    def _sub_copy(r, s, k, sub):
        return pltpu.make_async_copy(
            x_hbm.at[r, pl.ds(k * CH + sub * SUB, SUB), :],
            xbuf.at[s, k, pl.ds(sub * SUB, SUB), :],
            sems.at[s, k],
        )

    def _start_row(r, s):
        row_len = lens_ref[r]
        for k in range(NCH):
            t0 = k * CH

            @pl.when(t0 + CH <= row_len)
            def _start_full():
                pltpu.make_async_copy(
                    x_hbm.at[r, pl.ds(t0, CH), :],
                    xbuf.at[s, k],
                    sems.at[s, k],
                ).start()

            @pl.when(jnp.logical_and(t0 < row_len, t0 + CH > row_len))
            def _start_partial():
                for sub in range(CH // SUB):
                    @pl.when(t0 + sub * SUB < row_len)
                    def _start_sub():
                        _sub_copy(r, s, k, sub).start()

    @pl.when(j == 0)
    def _prologue():
        _start_row(0, 0)

    @pl.when(j + 1 < B)
    def _prefetch_next():
        _start_row(j + 1, (j + 1) % 2)

    def _out_copy(r, s, k):
        return pltpu.make_async_copy(
            obuf.at[s, k],
            out_hbm.at[r, pl.ds(k * CH, CH), :],
            osems.at[s, k],
        )

    @pl.when(j >= 2)
    def _reclaim():
        for k in range(NCH):
            _out_copy(j - 2, slot, k).wait()

    def _dot(x):
        return (
            jnp.dot(x, wt_ref[...], preferred_element_type=jnp.float32)
            + b_ref[...]
        )

    def _wait_in(k):
        pltpu.make_async_copy(
            x_hbm.at[j, pl.ds(k * CH, CH), :],
            xbuf.at[slot, k],
            sems.at[slot, k],
        ).wait()

    for k in range(NCH):
        t0 = k * CH

        @pl.when(t0 >= length)
        def _full_pad():
            pltpu.make_async_copy(
                cbuf,
                out_hbm.at[j, pl.ds(t0, CH), :],
                osems.at[slot, k],
            ).start()

    for k in range(NCH):
        t0 = k * CH

        @pl.when(t0 + CH <= length)
        def _full_valid():
            _wait_in(k)
            obuf[slot, k] = _dot(xbuf[slot, k])
            _out_copy(j, slot, k).start()

        @pl.when(jnp.logical_and(t0 < length, t0 + CH > length))
        def _partial():
            for sub in range(CH // SUB):
                @pl.when(t0 + sub * SUB < length)
                def _wait_sub():
                    _sub_copy(j, slot, k, sub).wait()
            rows = t0 + jax.lax.broadcasted_iota(jnp.int32, (CH, D), 0)
            obuf[slot, k] = jnp.where(rows < length, _dot(xbuf[slot, k]), NEG_INF)
            _out_copy(j, slot, k).start()

    @pl.when(j == B - 1)
    def _epilogue():
        for k in range(NCH):
            _out_copy(j - 1, (j - 1) % 2, k).wait()
        for k in range(NCH):
            _out_copy(j, slot, k).wait()


@functools.partial(jax.jit, static_argnames=())
def _run(padded, lengths, wt, b2):
    grid_spec = pltpu.PrefetchScalarGridSpec(
        num_scalar_prefetch=1,
        grid=(B,),
        in_specs=[
            pl.BlockSpec(memory_space=pl.ANY),
            pl.BlockSpec((D, D), lambda j, lens: (0, 0)),
            pl.BlockSpec((1, D), lambda j, lens: (0, 0)),
        ],
        out_specs=pl.BlockSpec(memory_space=pl.ANY),
        scratch_shapes=[
            pltpu.VMEM((2, NCH, CH, D), jnp.float32),
            pltpu.VMEM((2, NCH, CH, D), jnp.float32),
            pltpu.VMEM((CH, D), jnp.float32),
            pltpu.SemaphoreType.DMA((2, NCH)),
            pltpu.SemaphoreType.DMA((2, NCH)),
        ],
    )
    out = pl.pallas_call(
        _body,
        grid_spec=grid_spec,
        out_shape=jax.ShapeDtypeStruct((B, T, D), jnp.float32),
        compiler_params=pltpu.CompilerParams(
            dimension_semantics=("arbitrary",),
        ),
    )(lengths, padded, wt, b2)
    return out


def kernel(padded, lengths, W, b):
    wt = W.T
    b2 = b.reshape(1, D)
    out = _run(padded, lengths.astype(jnp.int32), wt, b2)
    return out, lengths
